# baseline (speedup 1.0000x reference)
import functools

import jax
import jax.numpy as jnp
from jax import lax
from jax.experimental import pallas as pl
from jax.experimental.pallas import tpu as pltpu
from jax.experimental.pallas import tpu_sc as plsc

N = 10000
E = 320000
D = 128
NC = 2
NS = 16
NW = NC * NS
EPT = E // NW
K = 80
NCH = EPT // K
NPAIR = (NCH - 1) // 2
N_PAD = 10240
RPT = N_PAD // NS

_MESH = plsc.VectorSubcoreMesh(core_axis_name="c", subcore_axis_name="s")
_SC_PARAMS = pltpu.CompilerParams(needs_layout_passes=False)


def _worker(cid, sid):
    return cid * NS + sid


def _make_seg():
    out_type = [jax.ShapeDtypeStruct((NC, N_PAD, D), jnp.float32)]
    scratch = [
        pltpu.VMEM((EPT,), jnp.int32),
        pltpu.VMEM((NCH, K), jnp.int32),
        pltpu.VMEM((K, D), jnp.float32),
        pltpu.VMEM((K, D), jnp.float32),
        pltpu.VMEM_SHARED((N_PAD, D), jnp.float32),
        pltpu.SemaphoreType.DMA,
        pltpu.SemaphoreType.DMA,
    ]

    @functools.partial(pl.kernel, out_type=out_type, mesh=_MESH,
                       scratch_types=scratch, compiler_params=_SC_PARAMS)
    def seg(h_hbm, src_hbm, dst_hbm, znd_hbm, agg_out,
            sidx, didx, rows0, rows1, agg_sh, sem0, sem1):
        cid = lax.axis_index("c")
        sid = lax.axis_index("s")
        wid = _worker(cid, sid)

        pltpu.sync_copy(znd_hbm.at[pl.ds(sid * RPT, RPT)],
                        agg_sh.at[pl.ds(sid * RPT, RPT)])
        pltpu.sync_copy(src_hbm.at[wid], sidx)
        pltpu.sync_copy(dst_hbm.at[wid], didx)
        plsc.subcore_barrier()

        def gather(c, rows, sem):
            pltpu.async_copy(h_hbm.at[sidx.at[pl.ds(c * K, K)]], rows, sem)

        def wait(c, rows, sem):
            pltpu.make_async_copy(h_hbm.at[sidx.at[pl.ds(c * K, K)]],
                                  rows, sem).wait()

        def consume(c, rows):
            pltpu.sync_copy(rows, agg_sh.at[didx.at[c]], add=True)

        gather(0, rows0, sem0)

        def pair(i, carry):
            c0 = 2 * i
            wait(c0, rows0, sem0)
            gather(c0 + 1, rows1, sem1)
            consume(c0, rows0)
            wait(c0 + 1, rows1, sem1)
            gather(c0 + 2, rows0, sem0)
            consume(c0 + 1, rows1)
            return carry

        lax.fori_loop(0, NPAIR, pair, 0)
        wait(NCH - 1, rows0, sem0)
        consume(NCH - 1, rows0)

        plsc.subcore_barrier()
        pltpu.sync_copy(agg_sh.at[pl.ds(sid * RPT, RPT)],
                        agg_out.at[cid].at[pl.ds(sid * RPT, RPT)])

    return seg


@functools.partial(
    pl.kernel,
    out_type=jax.ShapeDtypeStruct((NW, N_PAD), jnp.float32),
    mesh=_MESH,
    compiler_params=_SC_PARAMS,
    scratch_types=[
        pltpu.VMEM((EPT,), jnp.int32),
        pltpu.VMEM((N_PAD,), jnp.float32),
    ],
)
def _deg_kernel(dst_hbm, zn_hbm, deg_out, didx, deg_v):
    cid = lax.axis_index("c")
    sid = lax.axis_index("s")
    wid = _worker(cid, sid)
    pltpu.sync_copy(dst_hbm.at[wid], didx)
    pltpu.sync_copy(zn_hbm, deg_v)
    ones = jnp.full((16,), 1.0, jnp.float32)

    def sub(j, carry):
        d16 = didx[pl.ds(j * 16, 16)]
        plsc.addupdate_scatter(deg_v, [d16], ones)
        return carry

    lax.fori_loop(0, EPT // 16, sub, 0)
    pltpu.sync_copy(deg_v, deg_out.at[wid])


_seg = _make_seg()


_BN = 2000


def _dense_body(relu, out_dtype, h_ref, a0_ref, a1_ref, degp_ref, ws_ref,
                wn_ref, b_ref, o_ref):
    deg = jnp.sum(degp_ref[...], axis=1)
    r = 1.0 / jnp.clip(deg, 1.0, None)
    hn = (a0_ref[...] + a1_ref[...]) * r[:, None]
    y = (jnp.dot(h_ref[...], ws_ref[...], preferred_element_type=jnp.float32)
         + jnp.dot(hn, wn_ref[...], preferred_element_type=jnp.float32)
         + b_ref[...])
    y = jnp.maximum(y, 0.0) if relu else y
    o_ref[...] = y.astype(out_dtype)


def _dense(h, a0, a1, degp, ws, wn, b, relu, out_dtype=jnp.float32):
    grid = (N // _BN,)
    row_blk = pl.BlockSpec((_BN, D), lambda i: (i, 0))
    return pl.pallas_call(
        functools.partial(_dense_body, relu, out_dtype),
        grid=grid,
        in_specs=[
            row_blk,
            row_blk,
            row_blk,
            pl.BlockSpec((_BN, NW), lambda i: (i, 0)),
            pl.BlockSpec((D, D), lambda i: (0, 0)),
            pl.BlockSpec((D, D), lambda i: (0, 0)),
            pl.BlockSpec((1, D), lambda i: (0, 0)),
        ],
        out_specs=row_blk,
        out_shape=jax.ShapeDtypeStruct((N, D), out_dtype),
    )(h, a0, a1, degp, ws, wn, b.reshape(1, D))


@functools.partial(
    pl.kernel,
    out_type=[jax.ShapeDtypeStruct((E,), jnp.float32),
              jax.ShapeDtypeStruct((E,), jnp.float32)],
    mesh=_MESH,
    compiler_params=_SC_PARAMS,
    scratch_types=[
        pltpu.VMEM((EPT,), jnp.int32),
        pltpu.VMEM((EPT,), jnp.int32),
        pltpu.VMEM((K, D), jnp.float32),
        pltpu.VMEM((K, D), jnp.float32),
        pltpu.VMEM((K, D), jnp.float32),
        pltpu.VMEM((K, D), jnp.float32),
        pltpu.VMEM((EPT,), jnp.float32),
        pltpu.SemaphoreType.DMA,
        pltpu.SemaphoreType.DMA,
        pltpu.SemaphoreType.DMA,
        pltpu.SemaphoreType.DMA,
    ],
)
def _edge_dots(h_hbm, psrc_hbm, pdst_hbm, nsrc_hbm, ndst_hbm,
               pos_out, neg_out,
               uidx, vidx, urows0, vrows0, urows1, vrows1, scores,
               sem_u0, sem_v0, sem_u1, sem_v1):
    cid = lax.axis_index("c")
    sid = lax.axis_index("s")
    wid = _worker(cid, sid)

    lane0 = lax.iota(jnp.int32, 16) == 0

    for src_hbm, dst_hbm, out_hbm in ((psrc_hbm, pdst_hbm, pos_out),
                                      (nsrc_hbm, ndst_hbm, neg_out)):
        pltpu.sync_copy(src_hbm.at[wid], uidx)
        pltpu.sync_copy(dst_hbm.at[wid], vidx)

        def gather(c, urows, vrows, sem_u, sem_v):
            pltpu.async_copy(h_hbm.at[uidx.at[pl.ds(c * K, K)]], urows, sem_u)
            pltpu.async_copy(h_hbm.at[vidx.at[pl.ds(c * K, K)]], vrows, sem_v)

        def wait(c, urows, vrows, sem_u, sem_v):
            pltpu.make_async_copy(h_hbm.at[uidx.at[pl.ds(c * K, K)]],
                                  urows, sem_u).wait()
            pltpu.make_async_copy(h_hbm.at[vidx.at[pl.ds(c * K, K)]],
                                  vrows, sem_v).wait()

        def consume(c, urows, vrows):
            def edge(e, carry2):
                acc = urows[e, pl.ds(0, 16)] * vrows[e, pl.ds(0, 16)]
                for j in range(1, D // 16):
                    acc += (urows[e, pl.ds(j * 16, 16)]
                            * vrows[e, pl.ds(j * 16, 16)])
                s = jnp.sum(acc)
                plsc.store_scatter(scores,
                                   [jnp.full((16,), c * K + e, jnp.int32)],
                                   jnp.full((16,), s, jnp.float32),
                                   mask=lane0)
                return carry2

            lax.fori_loop(0, K, edge, 0, unroll=2)

        gather(0, urows0, vrows0, sem_u0, sem_v0)

        def pair(i, carry):
            c0 = 2 * i
            wait(c0, urows0, vrows0, sem_u0, sem_v0)
            gather(c0 + 1, urows1, vrows1, sem_u1, sem_v1)
            consume(c0, urows0, vrows0)
            wait(c0 + 1, urows1, vrows1, sem_u1, sem_v1)
            gather(c0 + 2, urows0, vrows0, sem_u0, sem_v0)
            consume(c0 + 1, urows1, vrows1)
            return carry

        lax.fori_loop(0, NPAIR, pair, 0)
        wait(NCH - 1, urows0, vrows0, sem_u0, sem_v0)
        consume(NCH - 1, urows0, vrows0)
        pltpu.sync_copy(scores, out_hbm.at[pl.ds(wid * EPT, EPT)])


def kernel(x, pos_edge_index, neg_edge_index, W_self1, W_neigh1, b1,
           W_self2, W_neigh2, b2):
    psrc = pos_edge_index[0].reshape(NW, EPT)
    pdst3 = pos_edge_index[1].reshape(NW, NCH, K)
    pdst = pos_edge_index[1].reshape(NW, EPT)
    nsrc = neg_edge_index[0].reshape(NW, EPT)
    ndst = neg_edge_index[1].reshape(NW, EPT)
    znd = jnp.zeros((N_PAD, D), jnp.float32)
    zn = jnp.zeros((N_PAD,), jnp.float32)

    agg1 = _seg(x, psrc, pdst3, znd)
    if isinstance(agg1, (list, tuple)):
        agg1 = agg1[0]
    degp = _deg_kernel(pdst, zn)
    degp = degp.T
    h1 = _dense(x, agg1[0], agg1[1], degp, W_self1, W_neigh1, b1, relu=True)
    agg2 = _seg(h1, psrc, pdst3, znd)
    if isinstance(agg2, (list, tuple)):
        agg2 = agg2[0]
    h2 = _dense(h1, agg2[0], agg2[1], degp, W_self2, W_neigh2, b2,
                relu=False)
    pos_s, neg_s = _edge_dots(h2, psrc, pdst, nsrc, ndst)
    return pos_s.reshape(E, 1), neg_s.reshape(E, 1)


# speedup vs baseline: 1.0265x; 1.0265x over previous
import functools

import jax
import jax.numpy as jnp
from jax import lax
from jax.experimental import pallas as pl
from jax.experimental.pallas import tpu as pltpu
from jax.experimental.pallas import tpu_sc as plsc

N = 10000
E = 320000
D = 128
NC = 2
NS = 16
NW = NC * NS
EPT = E // NW
K = 80
NCH = EPT // K
NPAIR = (NCH - 1) // 2
N_PAD = 10240
RPT = N_PAD // NS

_MESH = plsc.VectorSubcoreMesh(core_axis_name="c", subcore_axis_name="s")
_SC_PARAMS = pltpu.CompilerParams(needs_layout_passes=False)


def _worker(cid, sid):
    return cid * NS + sid


def _make_seg():
    out_type = [jax.ShapeDtypeStruct((NC, N_PAD, D), jnp.float32)]
    scratch = [
        pltpu.VMEM((EPT,), jnp.int32),
        pltpu.VMEM((NCH, K), jnp.int32),
        pltpu.VMEM((K, D), jnp.float32),
        pltpu.VMEM((K, D), jnp.float32),
        pltpu.VMEM_SHARED((N_PAD, D), jnp.float32),
        pltpu.SemaphoreType.DMA,
        pltpu.SemaphoreType.DMA,
    ]

    @functools.partial(pl.kernel, out_type=out_type, mesh=_MESH,
                       scratch_types=scratch, compiler_params=_SC_PARAMS)
    def seg(h_hbm, src_hbm, dst_hbm, znd_hbm, agg_out,
            sidx, didx, rows0, rows1, agg_sh, sem0, sem1):
        cid = lax.axis_index("c")
        sid = lax.axis_index("s")
        wid = _worker(cid, sid)

        pltpu.sync_copy(znd_hbm.at[pl.ds(sid * RPT, RPT)],
                        agg_sh.at[pl.ds(sid * RPT, RPT)])
        pltpu.sync_copy(src_hbm.at[wid], sidx)
        pltpu.sync_copy(dst_hbm.at[wid], didx)
        plsc.subcore_barrier()

        def gather(c, rows, sem):
            pltpu.async_copy(h_hbm.at[sidx.at[pl.ds(c * K, K)]], rows, sem)

        def wait(c, rows, sem):
            pltpu.make_async_copy(h_hbm.at[sidx.at[pl.ds(c * K, K)]],
                                  rows, sem).wait()

        def consume(c, rows):
            pltpu.sync_copy(rows, agg_sh.at[didx.at[c]], add=True)

        gather(0, rows0, sem0)

        def pair(i, carry):
            c0 = 2 * i
            wait(c0, rows0, sem0)
            gather(c0 + 1, rows1, sem1)
            consume(c0, rows0)
            wait(c0 + 1, rows1, sem1)
            gather(c0 + 2, rows0, sem0)
            consume(c0 + 1, rows1)
            return carry

        lax.fori_loop(0, NPAIR, pair, 0)
        wait(NCH - 1, rows0, sem0)
        consume(NCH - 1, rows0)

        plsc.subcore_barrier()
        pltpu.sync_copy(agg_sh.at[pl.ds(sid * RPT, RPT)],
                        agg_out.at[cid].at[pl.ds(sid * RPT, RPT)])

    return seg


@functools.partial(
    pl.kernel,
    out_type=jax.ShapeDtypeStruct((NW, N_PAD), jnp.float32),
    mesh=_MESH,
    compiler_params=_SC_PARAMS,
    scratch_types=[
        pltpu.VMEM((EPT,), jnp.int32),
        pltpu.VMEM((N_PAD,), jnp.float32),
    ],
)
def _deg_kernel(dst_hbm, zn_hbm, deg_out, didx, deg_v):
    cid = lax.axis_index("c")
    sid = lax.axis_index("s")
    wid = _worker(cid, sid)
    pltpu.sync_copy(dst_hbm.at[wid], didx)
    pltpu.sync_copy(zn_hbm, deg_v)
    ones = jnp.full((16,), 1.0, jnp.float32)

    def sub(j, carry):
        d16 = didx[pl.ds(j * 16, 16)]
        plsc.addupdate_scatter(deg_v, [d16], ones)
        return carry

    lax.fori_loop(0, EPT // 16, sub, 0)
    pltpu.sync_copy(deg_v, deg_out.at[wid])


_seg = _make_seg()


_BN = 2000


def _dense_body(relu, out_dtype, h_ref, a0_ref, a1_ref, degp_ref, ws_ref,
                wn_ref, b_ref, o_ref):
    deg = jnp.sum(degp_ref[...], axis=1)
    r = 1.0 / jnp.clip(deg, 1.0, None)
    hn = (a0_ref[...] + a1_ref[...]) * r[:, None]
    y = (jnp.dot(h_ref[...], ws_ref[...], preferred_element_type=jnp.float32)
         + jnp.dot(hn, wn_ref[...], preferred_element_type=jnp.float32)
         + b_ref[...])
    y = jnp.maximum(y, 0.0) if relu else y
    o_ref[...] = y.astype(out_dtype)


def _dense(h, a0, a1, degp, ws, wn, b, relu, out_dtype=jnp.float32):
    grid = (N // _BN,)
    row_blk = pl.BlockSpec((_BN, D), lambda i: (i, 0))
    return pl.pallas_call(
        functools.partial(_dense_body, relu, out_dtype),
        grid=grid,
        in_specs=[
            row_blk,
            row_blk,
            row_blk,
            pl.BlockSpec((_BN, NW), lambda i: (i, 0)),
            pl.BlockSpec((D, D), lambda i: (0, 0)),
            pl.BlockSpec((D, D), lambda i: (0, 0)),
            pl.BlockSpec((1, D), lambda i: (0, 0)),
        ],
        out_specs=row_blk,
        out_shape=jax.ShapeDtypeStruct((N, D), out_dtype),
    )(h, a0, a1, degp, ws, wn, b.reshape(1, D))


@functools.partial(
    pl.kernel,
    out_type=[jax.ShapeDtypeStruct((E,), jnp.float32),
              jax.ShapeDtypeStruct((E,), jnp.float32)],
    mesh=_MESH,
    compiler_params=_SC_PARAMS,
    scratch_types=[
        pltpu.VMEM((EPT,), jnp.int32),
        pltpu.VMEM((EPT,), jnp.int32),
        pltpu.VMEM((K, D), jnp.float32),
        pltpu.VMEM((K, D), jnp.float32),
        pltpu.VMEM((K, D), jnp.float32),
        pltpu.VMEM((K, D), jnp.float32),
        pltpu.VMEM((EPT,), jnp.float32),
        pltpu.VMEM((K, 16), jnp.float32),
        pltpu.SemaphoreType.DMA,
        pltpu.SemaphoreType.DMA,
        pltpu.SemaphoreType.DMA,
        pltpu.SemaphoreType.DMA,
    ],
)
def _edge_dots(h_hbm, psrc_hbm, pdst_hbm, nsrc_hbm, ndst_hbm,
               pos_out, neg_out,
               uidx, vidx, urows0, vrows0, urows1, vrows1, scores, part,
               sem_u0, sem_v0, sem_u1, sem_v1):
    cid = lax.axis_index("c")
    sid = lax.axis_index("s")
    wid = _worker(cid, sid)

    iota16 = lax.iota(jnp.int32, 16)

    for src_hbm, dst_hbm, out_hbm in ((psrc_hbm, pdst_hbm, pos_out),
                                      (nsrc_hbm, ndst_hbm, neg_out)):
        pltpu.sync_copy(src_hbm.at[wid], uidx)
        pltpu.sync_copy(dst_hbm.at[wid], vidx)

        def gather(c, urows, vrows, sem_u, sem_v):
            pltpu.async_copy(h_hbm.at[uidx.at[pl.ds(c * K, K)]], urows, sem_u)
            pltpu.async_copy(h_hbm.at[vidx.at[pl.ds(c * K, K)]], vrows, sem_v)

        def wait(c, urows, vrows, sem_u, sem_v):
            pltpu.make_async_copy(h_hbm.at[uidx.at[pl.ds(c * K, K)]],
                                  urows, sem_u).wait()
            pltpu.make_async_copy(h_hbm.at[vidx.at[pl.ds(c * K, K)]],
                                  vrows, sem_v).wait()

        def consume(c, urows, vrows):
            def edge(e, carry2):
                acc = urows[e, pl.ds(0, 16)] * vrows[e, pl.ds(0, 16)]
                for j in range(1, D // 16):
                    acc += (urows[e, pl.ds(j * 16, 16)]
                            * vrows[e, pl.ds(j * 16, 16)])
                part[e, pl.ds(0, 16)] = acc
                return carry2

            lax.fori_loop(0, K, edge, 0, unroll=2)

            def group(g, carry2):
                rows16 = g * 16 + iota16
                acc2 = plsc.load_gather(part,
                                        [rows16, jnp.zeros((16,), jnp.int32)])
                for l in range(1, 16):
                    acc2 += plsc.load_gather(part,
                                             [rows16,
                                              jnp.full((16,), l, jnp.int32)])
                scores[pl.ds(c * K + g * 16, 16)] = acc2
                return carry2

            lax.fori_loop(0, K // 16, group, 0)

        gather(0, urows0, vrows0, sem_u0, sem_v0)

        def pair(i, carry):
            c0 = 2 * i
            wait(c0, urows0, vrows0, sem_u0, sem_v0)
            gather(c0 + 1, urows1, vrows1, sem_u1, sem_v1)
            consume(c0, urows0, vrows0)
            wait(c0 + 1, urows1, vrows1, sem_u1, sem_v1)
            gather(c0 + 2, urows0, vrows0, sem_u0, sem_v0)
            consume(c0 + 1, urows1, vrows1)
            return carry

        lax.fori_loop(0, NPAIR, pair, 0)
        wait(NCH - 1, urows0, vrows0, sem_u0, sem_v0)
        consume(NCH - 1, urows0, vrows0)
        pltpu.sync_copy(scores, out_hbm.at[pl.ds(wid * EPT, EPT)])


def kernel(x, pos_edge_index, neg_edge_index, W_self1, W_neigh1, b1,
           W_self2, W_neigh2, b2):
    psrc = pos_edge_index[0].reshape(NW, EPT)
    pdst3 = pos_edge_index[1].reshape(NW, NCH, K)
    pdst = pos_edge_index[1].reshape(NW, EPT)
    nsrc = neg_edge_index[0].reshape(NW, EPT)
    ndst = neg_edge_index[1].reshape(NW, EPT)
    znd = jnp.zeros((N_PAD, D), jnp.float32)
    zn = jnp.zeros((N_PAD,), jnp.float32)

    agg1 = _seg(x, psrc, pdst3, znd)
    if isinstance(agg1, (list, tuple)):
        agg1 = agg1[0]
    degp = _deg_kernel(pdst, zn)
    degp = degp.T
    h1 = _dense(x, agg1[0], agg1[1], degp, W_self1, W_neigh1, b1, relu=True)
    agg2 = _seg(h1, psrc, pdst3, znd)
    if isinstance(agg2, (list, tuple)):
        agg2 = agg2[0]
    h2 = _dense(h1, agg2[0], agg2[1], degp, W_self2, W_neigh2, b2,
                relu=False)
    pos_s, neg_s = _edge_dots(h2, psrc, pdst, nsrc, ndst)
    return pos_s.reshape(E, 1), neg_s.reshape(E, 1)


# speedup vs baseline: 1.0302x; 1.0037x over previous
import functools

import jax
import jax.numpy as jnp
from jax import lax
from jax.experimental import pallas as pl
from jax.experimental.pallas import tpu as pltpu
from jax.experimental.pallas import tpu_sc as plsc

N = 10000
E = 320000
D = 128
NC = 2
NS = 16
NW = NC * NS
EPT = E // NW
K = 80
NCH = EPT // K
NPAIR = (NCH - 1) // 2
N_PAD = 10240
RPT = N_PAD // NS

_MESH = plsc.VectorSubcoreMesh(core_axis_name="c", subcore_axis_name="s")
_SC_PARAMS = pltpu.CompilerParams(needs_layout_passes=False)


def _worker(cid, sid):
    return cid * NS + sid


def _make_seg():
    out_type = [jax.ShapeDtypeStruct((NC, N_PAD, D), jnp.float32)]
    scratch = [
        pltpu.VMEM((EPT,), jnp.int32),
        pltpu.VMEM((NCH, K), jnp.int32),
        pltpu.VMEM((K, D), jnp.float32),
        pltpu.VMEM((K, D), jnp.float32),
        pltpu.VMEM_SHARED((N_PAD, D), jnp.float32),
        pltpu.SemaphoreType.DMA,
        pltpu.SemaphoreType.DMA,
    ]

    @functools.partial(pl.kernel, out_type=out_type, mesh=_MESH,
                       scratch_types=scratch, compiler_params=_SC_PARAMS)
    def seg(h_hbm, src_hbm, dst_hbm, znd_hbm, agg_out,
            sidx, didx, rows0, rows1, agg_sh, sem0, sem1):
        cid = lax.axis_index("c")
        sid = lax.axis_index("s")
        wid = _worker(cid, sid)

        pltpu.sync_copy(znd_hbm.at[pl.ds(sid * RPT, RPT)],
                        agg_sh.at[pl.ds(sid * RPT, RPT)])
        pltpu.sync_copy(src_hbm.at[wid], sidx)
        pltpu.sync_copy(dst_hbm.at[wid], didx)
        plsc.subcore_barrier()

        def gather(c, rows, sem):
            pltpu.async_copy(h_hbm.at[sidx.at[pl.ds(c * K, K)]], rows, sem)

        def wait(c, rows, sem):
            pltpu.make_async_copy(h_hbm.at[sidx.at[pl.ds(c * K, K)]],
                                  rows, sem).wait()

        def consume(c, rows):
            pltpu.sync_copy(rows, agg_sh.at[didx.at[c]], add=True)

        gather(0, rows0, sem0)

        def pair(i, carry):
            c0 = 2 * i
            wait(c0, rows0, sem0)
            gather(c0 + 1, rows1, sem1)
            consume(c0, rows0)
            wait(c0 + 1, rows1, sem1)
            gather(c0 + 2, rows0, sem0)
            consume(c0 + 1, rows1)
            return carry

        lax.fori_loop(0, NPAIR, pair, 0)
        wait(NCH - 1, rows0, sem0)
        consume(NCH - 1, rows0)

        plsc.subcore_barrier()
        pltpu.sync_copy(agg_sh.at[pl.ds(sid * RPT, RPT)],
                        agg_out.at[cid].at[pl.ds(sid * RPT, RPT)])

    return seg


@functools.partial(
    pl.kernel,
    out_type=jax.ShapeDtypeStruct((NW, N_PAD), jnp.float32),
    mesh=_MESH,
    compiler_params=_SC_PARAMS,
    scratch_types=[
        pltpu.VMEM((EPT,), jnp.int32),
        pltpu.VMEM((N_PAD,), jnp.float32),
    ],
)
def _deg_kernel(dst_hbm, zn_hbm, deg_out, didx, deg_v):
    cid = lax.axis_index("c")
    sid = lax.axis_index("s")
    wid = _worker(cid, sid)
    pltpu.sync_copy(dst_hbm.at[wid], didx)
    pltpu.sync_copy(zn_hbm, deg_v)
    ones = jnp.full((16,), 1.0, jnp.float32)

    def sub(j, carry):
        d16 = didx[pl.ds(j * 16, 16)]
        plsc.addupdate_scatter(deg_v, [d16], ones)
        return carry

    lax.fori_loop(0, EPT // 16, sub, 0)
    pltpu.sync_copy(deg_v, deg_out.at[wid])


_seg = _make_seg()


_BN = 2000


def _dense_body(relu, out_dtype, h_ref, a0_ref, a1_ref, degp_ref, ws_ref,
                wn_ref, b_ref, o_ref):
    deg = jnp.sum(degp_ref[...], axis=1)
    r = 1.0 / jnp.clip(deg, 1.0, None)
    hn = (a0_ref[...] + a1_ref[...]) * r[:, None]
    y = (jnp.dot(h_ref[...], ws_ref[...], preferred_element_type=jnp.float32)
         + jnp.dot(hn, wn_ref[...], preferred_element_type=jnp.float32)
         + b_ref[...])
    y = jnp.maximum(y, 0.0) if relu else y
    o_ref[...] = y.astype(out_dtype)


def _dense(h, a0, a1, degp, ws, wn, b, relu, out_dtype=jnp.float32):
    grid = (N // _BN,)
    row_blk = pl.BlockSpec((_BN, D), lambda i: (i, 0))
    return pl.pallas_call(
        functools.partial(_dense_body, relu, out_dtype),
        grid=grid,
        in_specs=[
            row_blk,
            row_blk,
            row_blk,
            pl.BlockSpec((_BN, NW), lambda i: (i, 0)),
            pl.BlockSpec((D, D), lambda i: (0, 0)),
            pl.BlockSpec((D, D), lambda i: (0, 0)),
            pl.BlockSpec((1, D), lambda i: (0, 0)),
        ],
        out_specs=row_blk,
        out_shape=jax.ShapeDtypeStruct((N, D), out_dtype),
    )(h, a0, a1, degp, ws, wn, b.reshape(1, D))


_BA = 1000
_BV = 2048


def _gram_body(he_ref, ho_ref, hv_ref, o_ref):
    hv = hv_ref[...].astype(jnp.bfloat16)
    ye = jax.lax.dot_general(he_ref[...].astype(jnp.bfloat16), hv,
                             (((1,), (1,)), ((), ())),
                             preferred_element_type=jnp.float32)
    yo = jax.lax.dot_general(ho_ref[...].astype(jnp.bfloat16), hv,
                             (((1,), (1,)), ((), ())),
                             preferred_element_type=jnp.float32)
    ze = jax.lax.bitcast_convert_type(ye.astype(jnp.bfloat16),
                                      jnp.uint16).astype(jnp.uint32)
    zo = jax.lax.bitcast_convert_type(yo.astype(jnp.bfloat16),
                                      jnp.uint16).astype(jnp.uint32)
    o_ref[...] = (ze | (zo << 16)).astype(jnp.int32)


def _gram(he, ho, h2p):
    return pl.pallas_call(
        _gram_body,
        grid=(N // 2 // _BA, N_PAD // _BV),
        in_specs=[
            pl.BlockSpec((_BA, D), lambda i, j: (i, 0)),
            pl.BlockSpec((_BA, D), lambda i, j: (i, 0)),
            pl.BlockSpec((_BV, D), lambda i, j: (j, 0)),
        ],
        out_specs=pl.BlockSpec((_BA, _BV), lambda i, j: (i, j)),
        out_shape=jax.ShapeDtypeStruct((N // 2, N_PAD), jnp.int32),
    )(he, ho, h2p)


@functools.partial(
    pl.kernel,
    out_type=[jax.ShapeDtypeStruct((E,), jnp.float32),
              jax.ShapeDtypeStruct((E,), jnp.float32)],
    mesh=_MESH,
    compiler_params=_SC_PARAMS,
    scratch_types=[
        pltpu.VMEM((EPT,), jnp.int32),
        pltpu.VMEM((EPT,), jnp.int32),
        pltpu.VMEM((K,), jnp.int32),
        pltpu.VMEM((K,), jnp.int32),
        pltpu.VMEM((K,), jnp.int32),
        pltpu.VMEM((K,), jnp.int32),
        pltpu.VMEM((EPT,), jnp.float32),
        pltpu.SemaphoreType.DMA,
        pltpu.SemaphoreType.DMA,
    ],
)
def _edge_dots(g_hbm, psrc_hbm, pdst_hbm, nsrc_hbm, ndst_hbm,
               pos_out, neg_out,
               uidx, vidx, widx0, widx1, words0, words1, scores,
               sem0, sem1):
    cid = lax.axis_index("c")
    sid = lax.axis_index("s")
    wid = _worker(cid, sid)

    for src_hbm, dst_hbm, out_hbm in ((psrc_hbm, pdst_hbm, pos_out),
                                      (nsrc_hbm, ndst_hbm, neg_out)):
        pltpu.sync_copy(src_hbm.at[wid], uidx)
        pltpu.sync_copy(dst_hbm.at[wid], vidx)

        def windex(c, widx):
            def grp(g, carry):
                u = uidx[pl.ds(c * K + g * 16, 16)]
                v = vidx[pl.ds(c * K + g * 16, 16)]
                widx[pl.ds(g * 16, 16)] = (
                    lax.shift_right_logical(u, 1) * N_PAD + v)
                return carry
            lax.fori_loop(0, K // 16, grp, 0, unroll=True)

        def gather(widx, words, sem):
            pltpu.async_copy(g_hbm.at[widx], words, sem)

        def wait(widx, words, sem):
            pltpu.make_async_copy(g_hbm.at[widx], words, sem).wait()

        def consume(c, words):
            def grp(g, carry):
                w = words[pl.ds(g * 16, 16)]
                u = uidx[pl.ds(c * K + g * 16, 16)]
                b = plsc.bitcast(w, jnp.bfloat16)
                lo, hi = plsc.unpack(b, format=plsc.PackFormat.INTERLEAVED)
                odd = lax.bitwise_and(u, 1) == 1
                scores[pl.ds(c * K + g * 16, 16)] = jnp.where(odd, hi, lo)
                return carry
            lax.fori_loop(0, K // 16, grp, 0, unroll=True)

        windex(0, widx0)
        gather(widx0, words0, sem0)

        def pair(i, carry):
            c0 = 2 * i
            windex(c0 + 1, widx1)
            wait(widx0, words0, sem0)
            gather(widx1, words1, sem1)
            consume(c0, words0)
            windex(c0 + 2, widx0)
            wait(widx1, words1, sem1)
            gather(widx0, words0, sem0)
            consume(c0 + 1, words1)
            return carry

        lax.fori_loop(0, NPAIR, pair, 0)
        wait(widx0, words0, sem0)
        consume(NCH - 1, words0)
        pltpu.sync_copy(scores, out_hbm.at[pl.ds(wid * EPT, EPT)])


def kernel(x, pos_edge_index, neg_edge_index, W_self1, W_neigh1, b1,
           W_self2, W_neigh2, b2):
    psrc = pos_edge_index[0].reshape(NW, EPT)
    pdst3 = pos_edge_index[1].reshape(NW, NCH, K)
    pdst = pos_edge_index[1].reshape(NW, EPT)
    nsrc = neg_edge_index[0].reshape(NW, EPT)
    ndst = neg_edge_index[1].reshape(NW, EPT)
    znd = jnp.zeros((N_PAD, D), jnp.float32)
    zn = jnp.zeros((N_PAD,), jnp.float32)

    agg1 = _seg(x, psrc, pdst3, znd)
    if isinstance(agg1, (list, tuple)):
        agg1 = agg1[0]
    degp = _deg_kernel(pdst, zn)
    degp = degp.T
    h1 = _dense(x, agg1[0], agg1[1], degp, W_self1, W_neigh1, b1, relu=True)
    agg2 = _seg(h1, psrc, pdst3, znd)
    if isinstance(agg2, (list, tuple)):
        agg2 = agg2[0]
    h2 = _dense(h1, agg2[0], agg2[1], degp, W_self2, W_neigh2, b2,
                relu=False)
    h2p = jnp.pad(h2, ((0, N_PAD - N), (0, 0)))
    gp = _gram(h2[0::2], h2[1::2], h2p)
    pos_s, neg_s = _edge_dots(gp.reshape(-1), psrc, pdst, nsrc, ndst)
    return pos_s.reshape(E, 1), neg_s.reshape(E, 1)


# speedup vs baseline: 1.0548x; 1.0238x over previous
import functools

import jax
import jax.numpy as jnp
from jax import lax
from jax.experimental import pallas as pl
from jax.experimental.pallas import tpu as pltpu
from jax.experimental.pallas import tpu_sc as plsc

N = 10000
E = 320000
D = 128
NC = 2
NS = 16
NW = NC * NS
EPT = E // NW
K = 80
NCH = EPT // K
NPAIR = (NCH - 1) // 2
N_PAD = 10240
RPT = N_PAD // NS

_MESH = plsc.VectorSubcoreMesh(core_axis_name="c", subcore_axis_name="s")
_SC_PARAMS = pltpu.CompilerParams(needs_layout_passes=False)


def _worker(cid, sid):
    return cid * NS + sid


def _make_seg():
    out_type = [jax.ShapeDtypeStruct((NC, N_PAD, D), jnp.float32)]
    scratch = [
        pltpu.VMEM((EPT,), jnp.int32),
        pltpu.VMEM((NCH, K), jnp.int32),
        pltpu.VMEM((K, D), jnp.float32),
        pltpu.VMEM((K, D), jnp.float32),
        pltpu.VMEM_SHARED((N_PAD, D), jnp.float32),
        pltpu.SemaphoreType.DMA,
        pltpu.SemaphoreType.DMA,
    ]

    @functools.partial(pl.kernel, out_type=out_type, mesh=_MESH,
                       scratch_types=scratch, compiler_params=_SC_PARAMS)
    def seg(h_hbm, src_hbm, dst_hbm, znd_hbm, agg_out,
            sidx, didx, rows0, rows1, agg_sh, sem0, sem1):
        cid = lax.axis_index("c")
        sid = lax.axis_index("s")
        wid = _worker(cid, sid)

        pltpu.sync_copy(znd_hbm.at[pl.ds(sid * RPT, RPT)],
                        agg_sh.at[pl.ds(sid * RPT, RPT)])
        pltpu.sync_copy(src_hbm.at[wid], sidx)
        pltpu.sync_copy(dst_hbm.at[wid], didx)
        plsc.subcore_barrier()

        def gather(c, rows, sem):
            pltpu.async_copy(h_hbm.at[sidx.at[pl.ds(c * K, K)]], rows, sem)

        def wait(c, rows, sem):
            pltpu.make_async_copy(h_hbm.at[sidx.at[pl.ds(c * K, K)]],
                                  rows, sem).wait()

        def consume(c, rows):
            pltpu.sync_copy(rows, agg_sh.at[didx.at[c]], add=True)

        gather(0, rows0, sem0)

        def pair(i, carry):
            c0 = 2 * i
            wait(c0, rows0, sem0)
            gather(c0 + 1, rows1, sem1)
            consume(c0, rows0)
            wait(c0 + 1, rows1, sem1)
            gather(c0 + 2, rows0, sem0)
            consume(c0 + 1, rows1)
            return carry

        lax.fori_loop(0, NPAIR, pair, 0)
        wait(NCH - 1, rows0, sem0)
        consume(NCH - 1, rows0)

        plsc.subcore_barrier()
        pltpu.sync_copy(agg_sh.at[pl.ds(sid * RPT, RPT)],
                        agg_out.at[cid].at[pl.ds(sid * RPT, RPT)])

    return seg


@functools.partial(
    pl.kernel,
    out_type=jax.ShapeDtypeStruct((NW, N_PAD), jnp.float32),
    mesh=_MESH,
    compiler_params=_SC_PARAMS,
    scratch_types=[
        pltpu.VMEM((EPT,), jnp.int32),
        pltpu.VMEM((N_PAD,), jnp.float32),
    ],
)
def _deg_kernel(dst_hbm, zn_hbm, deg_out, didx, deg_v):
    cid = lax.axis_index("c")
    sid = lax.axis_index("s")
    wid = _worker(cid, sid)
    pltpu.sync_copy(dst_hbm.at[wid], didx)
    pltpu.sync_copy(zn_hbm, deg_v)
    ones = jnp.full((16,), 1.0, jnp.float32)

    def sub(j, carry):
        d16 = didx[pl.ds(j * 16, 16)]
        plsc.addupdate_scatter(deg_v, [d16], ones)
        return carry

    lax.fori_loop(0, EPT // 16, sub, 0)
    pltpu.sync_copy(deg_v, deg_out.at[wid])


_seg = _make_seg()


_BN = 2000


def _dense_body(relu, out_dtype, h_ref, a0_ref, a1_ref, degp_ref, ws_ref,
                wn_ref, b_ref, o_ref):
    deg = jnp.sum(degp_ref[...], axis=1)
    r = 1.0 / jnp.clip(deg, 1.0, None)
    hn = (a0_ref[...] + a1_ref[...]) * r[:, None]
    y = (jnp.dot(h_ref[...], ws_ref[...], preferred_element_type=jnp.float32)
         + jnp.dot(hn, wn_ref[...], preferred_element_type=jnp.float32)
         + b_ref[...])
    y = jnp.maximum(y, 0.0) if relu else y
    o_ref[...] = y.astype(out_dtype)


def _dense(h, a0, a1, degp, ws, wn, b, relu, out_dtype=jnp.float32):
    grid = (N // _BN,)
    row_blk = pl.BlockSpec((_BN, D), lambda i: (i, 0))
    return pl.pallas_call(
        functools.partial(_dense_body, relu, out_dtype),
        grid=grid,
        in_specs=[
            row_blk,
            row_blk,
            row_blk,
            pl.BlockSpec((_BN, NW), lambda i: (i, 0)),
            pl.BlockSpec((D, D), lambda i: (0, 0)),
            pl.BlockSpec((D, D), lambda i: (0, 0)),
            pl.BlockSpec((1, D), lambda i: (0, 0)),
        ],
        out_specs=row_blk,
        out_shape=jax.ShapeDtypeStruct((N, D), out_dtype),
    )(h, a0, a1, degp, ws, wn, b.reshape(1, D))


_BA = 1000
_BV = 2048


def _gram_body(hr_ref, hv_ref, o_ref):
    hv = hv_ref[...].astype(jnp.bfloat16)
    hr = hr_ref[...].astype(jnp.bfloat16)
    ye = jax.lax.dot_general(hr[:, 0, :], hv,
                             (((1,), (1,)), ((), ())),
                             preferred_element_type=jnp.float32)
    yo = jax.lax.dot_general(hr[:, 1, :], hv,
                             (((1,), (1,)), ((), ())),
                             preferred_element_type=jnp.float32)
    ze = jax.lax.bitcast_convert_type(ye.astype(jnp.bfloat16),
                                      jnp.uint16).astype(jnp.uint32)
    zo = jax.lax.bitcast_convert_type(yo.astype(jnp.bfloat16),
                                      jnp.uint16).astype(jnp.uint32)
    o_ref[...] = (ze | (zo << 16)).astype(jnp.int32)


def _gram(h2r, h2):
    return pl.pallas_call(
        _gram_body,
        grid=(N // 2 // _BA, N_PAD // _BV),
        in_specs=[
            pl.BlockSpec((_BA, 2, D), lambda i, j: (i, 0, 0)),
            pl.BlockSpec((_BV, D), lambda i, j: (j, 0)),
        ],
        out_specs=pl.BlockSpec((_BA, _BV), lambda i, j: (i, j)),
        out_shape=jax.ShapeDtypeStruct((N // 2, N_PAD), jnp.int32),
    )(h2r, h2)


@functools.partial(
    pl.kernel,
    out_type=[jax.ShapeDtypeStruct((E,), jnp.float32),
              jax.ShapeDtypeStruct((E,), jnp.float32)],
    mesh=_MESH,
    compiler_params=_SC_PARAMS,
    scratch_types=[
        pltpu.VMEM((EPT,), jnp.int32),
        pltpu.VMEM((EPT,), jnp.int32),
        pltpu.VMEM((K,), jnp.int32),
        pltpu.VMEM((K,), jnp.int32),
        pltpu.VMEM((K,), jnp.int32),
        pltpu.VMEM((K,), jnp.int32),
        pltpu.VMEM((EPT,), jnp.float32),
        pltpu.SemaphoreType.DMA,
        pltpu.SemaphoreType.DMA,
    ],
)
def _edge_dots(g_hbm, psrc_hbm, pdst_hbm, nsrc_hbm, ndst_hbm,
               pos_out, neg_out,
               uidx, vidx, widx0, widx1, words0, words1, scores,
               sem0, sem1):
    cid = lax.axis_index("c")
    sid = lax.axis_index("s")
    wid = _worker(cid, sid)

    for src_hbm, dst_hbm, out_hbm in ((psrc_hbm, pdst_hbm, pos_out),
                                      (nsrc_hbm, ndst_hbm, neg_out)):
        pltpu.sync_copy(src_hbm.at[wid], uidx)
        pltpu.sync_copy(dst_hbm.at[wid], vidx)

        def windex(c, widx):
            def grp(g, carry):
                u = uidx[pl.ds(c * K + g * 16, 16)]
                v = vidx[pl.ds(c * K + g * 16, 16)]
                widx[pl.ds(g * 16, 16)] = (
                    lax.shift_right_logical(u, 1) * N_PAD + v)
                return carry
            lax.fori_loop(0, K // 16, grp, 0, unroll=True)

        def gather(widx, words, sem):
            pltpu.async_copy(g_hbm.at[widx], words, sem)

        def wait(widx, words, sem):
            pltpu.make_async_copy(g_hbm.at[widx], words, sem).wait()

        def consume(c, words):
            def grp(g, carry):
                w = words[pl.ds(g * 16, 16)]
                u = uidx[pl.ds(c * K + g * 16, 16)]
                b = plsc.bitcast(w, jnp.bfloat16)
                lo, hi = plsc.unpack(b, format=plsc.PackFormat.INTERLEAVED)
                odd = lax.bitwise_and(u, 1) == 1
                scores[pl.ds(c * K + g * 16, 16)] = jnp.where(odd, hi, lo)
                return carry
            lax.fori_loop(0, K // 16, grp, 0, unroll=True)

        windex(0, widx0)
        gather(widx0, words0, sem0)

        def pair(i, carry):
            c0 = 2 * i
            windex(c0 + 1, widx1)
            wait(widx0, words0, sem0)
            gather(widx1, words1, sem1)
            consume(c0, words0)
            windex(c0 + 2, widx0)
            wait(widx1, words1, sem1)
            gather(widx0, words0, sem0)
            consume(c0 + 1, words1)
            return carry

        lax.fori_loop(0, NPAIR, pair, 0)
        wait(widx0, words0, sem0)
        consume(NCH - 1, words0)
        pltpu.sync_copy(scores, out_hbm.at[pl.ds(wid * EPT, EPT)])


def kernel(x, pos_edge_index, neg_edge_index, W_self1, W_neigh1, b1,
           W_self2, W_neigh2, b2):
    psrc = pos_edge_index[0].reshape(NW, EPT)
    pdst3 = pos_edge_index[1].reshape(NW, NCH, K)
    pdst = pos_edge_index[1].reshape(NW, EPT)
    nsrc = neg_edge_index[0].reshape(NW, EPT)
    ndst = neg_edge_index[1].reshape(NW, EPT)
    znd = jnp.zeros((N_PAD, D), jnp.float32)
    zn = jnp.zeros((N_PAD,), jnp.float32)

    agg1 = _seg(x, psrc, pdst3, znd)
    if isinstance(agg1, (list, tuple)):
        agg1 = agg1[0]
    degp = _deg_kernel(pdst, zn)
    degp = degp.T
    h1 = _dense(x, agg1[0], agg1[1], degp, W_self1, W_neigh1, b1, relu=True)
    agg2 = _seg(h1, psrc, pdst3, znd)
    if isinstance(agg2, (list, tuple)):
        agg2 = agg2[0]
    h2 = _dense(h1, agg2[0], agg2[1], degp, W_self2, W_neigh2, b2,
                relu=False)
    gp = _gram(h2.reshape(N // 2, 2, D), h2)
    pos_s, neg_s = _edge_dots(gp.reshape(-1), psrc, pdst, nsrc, ndst)
    return pos_s.reshape(E, 1), neg_s.reshape(E, 1)


# speedup vs baseline: 1.2299x; 1.1660x over previous
import functools

import jax
import jax.numpy as jnp
from jax import lax
from jax.experimental import pallas as pl
from jax.experimental.pallas import tpu as pltpu
from jax.experimental.pallas import tpu_sc as plsc

N = 10000
E = 320000
D = 128
NC = 2
NS = 16
NW = NC * NS
EPT = E // NW
K = 80
NCH = EPT // K
NPAIR = (NCH - 1) // 2
N_PAD = 10240
RPT = N_PAD // NS

_MESH = plsc.VectorSubcoreMesh(core_axis_name="c", subcore_axis_name="s")
_SC_PARAMS = pltpu.CompilerParams(needs_layout_passes=False)


def _worker(cid, sid):
    return cid * NS + sid


def _make_seg():
    out_type = [jax.ShapeDtypeStruct((NC, N_PAD, D), jnp.float32)]
    scratch = [
        pltpu.VMEM((EPT,), jnp.int32),
        pltpu.VMEM((NCH, K), jnp.int32),
        pltpu.VMEM((K, D), jnp.float32),
        pltpu.VMEM((K, D), jnp.float32),
        pltpu.VMEM_SHARED((N_PAD, D), jnp.float32),
        pltpu.SemaphoreType.DMA,
        pltpu.SemaphoreType.DMA,
    ]

    @functools.partial(pl.kernel, out_type=out_type, mesh=_MESH,
                       scratch_types=scratch, compiler_params=_SC_PARAMS)
    def seg(h_hbm, src_hbm, dst_hbm, znd_hbm, agg_out,
            sidx, didx, rows0, rows1, agg_sh, sem0, sem1):
        cid = lax.axis_index("c")
        sid = lax.axis_index("s")
        wid = _worker(cid, sid)

        pltpu.sync_copy(znd_hbm.at[pl.ds(sid * RPT, RPT)],
                        agg_sh.at[pl.ds(sid * RPT, RPT)])
        pltpu.sync_copy(src_hbm.at[wid], sidx)
        pltpu.sync_copy(dst_hbm.at[wid], didx)
        plsc.subcore_barrier()

        def gather(c, rows, sem):
            pltpu.async_copy(h_hbm.at[sidx.at[pl.ds(c * K, K)]], rows, sem)

        def wait(c, rows, sem):
            pltpu.make_async_copy(h_hbm.at[sidx.at[pl.ds(c * K, K)]],
                                  rows, sem).wait()

        def consume(c, rows):
            pltpu.sync_copy(rows, agg_sh.at[didx.at[c]], add=True)

        gather(0, rows0, sem0)

        def pair(i, carry):
            c0 = 2 * i
            wait(c0, rows0, sem0)
            gather(c0 + 1, rows1, sem1)
            consume(c0, rows0)
            wait(c0 + 1, rows1, sem1)
            gather(c0 + 2, rows0, sem0)
            consume(c0 + 1, rows1)
            return carry

        lax.fori_loop(0, NPAIR, pair, 0)
        wait(NCH - 1, rows0, sem0)
        consume(NCH - 1, rows0)

        plsc.subcore_barrier()
        pltpu.sync_copy(agg_sh.at[pl.ds(sid * RPT, RPT)],
                        agg_out.at[cid].at[pl.ds(sid * RPT, RPT)])

    return seg


@functools.partial(
    pl.kernel,
    out_type=jax.ShapeDtypeStruct((NW, N_PAD), jnp.float32),
    mesh=_MESH,
    compiler_params=_SC_PARAMS,
    scratch_types=[
        pltpu.VMEM((EPT,), jnp.int32),
        pltpu.VMEM((N_PAD,), jnp.float32),
    ],
)
def _deg_kernel(dst_hbm, zn_hbm, deg_out, didx, deg_v):
    cid = lax.axis_index("c")
    sid = lax.axis_index("s")
    wid = _worker(cid, sid)
    pltpu.sync_copy(dst_hbm.at[wid], didx)
    pltpu.sync_copy(zn_hbm, deg_v)
    ones = jnp.full((16,), 1.0, jnp.float32)

    def sub(j, carry):
        d16 = didx[pl.ds(j * 16, 16)]
        plsc.addupdate_scatter(deg_v, [d16], ones)
        return carry

    lax.fori_loop(0, EPT // 16, sub, 0)
    pltpu.sync_copy(deg_v, deg_out.at[wid])


_seg = _make_seg()


_BN = 2000


def _dense_body(relu, out_dtype, h_ref, a0_ref, a1_ref, degp_ref, ws_ref,
                wn_ref, b_ref, o_ref):
    deg = jnp.sum(degp_ref[...], axis=1)
    r = 1.0 / jnp.clip(deg, 1.0, None)
    hn = (a0_ref[...] + a1_ref[...]) * r[:, None]
    y = (jnp.dot(h_ref[...], ws_ref[...], preferred_element_type=jnp.float32)
         + jnp.dot(hn, wn_ref[...], preferred_element_type=jnp.float32)
         + b_ref[...])
    y = jnp.maximum(y, 0.0) if relu else y
    o_ref[...] = y.astype(out_dtype)


def _dense(h, a0, a1, degp, ws, wn, b, relu, out_dtype=jnp.float32):
    grid = (N // _BN,)
    row_blk = pl.BlockSpec((_BN, D), lambda i: (i, 0))
    return pl.pallas_call(
        functools.partial(_dense_body, relu, out_dtype),
        grid=grid,
        in_specs=[
            row_blk,
            row_blk,
            row_blk,
            pl.BlockSpec((_BN, NW), lambda i: (i, 0)),
            pl.BlockSpec((D, D), lambda i: (0, 0)),
            pl.BlockSpec((D, D), lambda i: (0, 0)),
            pl.BlockSpec((1, D), lambda i: (0, 0)),
        ],
        out_specs=row_blk,
        out_shape=jax.ShapeDtypeStruct((N, D), out_dtype),
    )(h, a0, a1, degp, ws, wn, b.reshape(1, D))


_BA = 125


def _gram_body(hr_ref, hv_ref, o_ref):
    hv = hv_ref[...].astype(jnp.bfloat16)
    hr = hr_ref[...].astype(jnp.bfloat16)
    ye = jax.lax.dot_general(hr[:, 0, :], hv,
                             (((1,), (1,)), ((), ())),
                             preferred_element_type=jnp.float32)
    yo = jax.lax.dot_general(hr[:, 1, :], hv,
                             (((1,), (1,)), ((), ())),
                             preferred_element_type=jnp.float32)
    ze = jax.lax.bitcast_convert_type(ye.astype(jnp.bfloat16),
                                      jnp.uint16).astype(jnp.uint32)
    zo = jax.lax.bitcast_convert_type(yo.astype(jnp.bfloat16),
                                      jnp.uint16).astype(jnp.uint32)
    packed = (ze | (zo << 16)).astype(jnp.int32)
    o_ref[...] = packed.reshape(_BA * N_PAD)


def _gram(h2r, h2):
    return pl.pallas_call(
        _gram_body,
        grid=(N // 2 // _BA,),
        in_specs=[
            pl.BlockSpec((_BA, 2, D), lambda i: (i, 0, 0)),
            pl.BlockSpec((N_PAD, D), lambda i: (0, 0)),
        ],
        out_specs=pl.BlockSpec((_BA * N_PAD,), lambda i: (i,)),
        out_shape=jax.ShapeDtypeStruct((N // 2 * N_PAD,), jnp.int32),
    )(h2r, h2)


@functools.partial(
    pl.kernel,
    out_type=[jax.ShapeDtypeStruct((E,), jnp.float32),
              jax.ShapeDtypeStruct((E,), jnp.float32)],
    mesh=_MESH,
    compiler_params=_SC_PARAMS,
    scratch_types=[
        pltpu.VMEM((EPT,), jnp.int32),
        pltpu.VMEM((EPT,), jnp.int32),
        pltpu.VMEM((K,), jnp.int32),
        pltpu.VMEM((K,), jnp.int32),
        pltpu.VMEM((K,), jnp.int32),
        pltpu.VMEM((K,), jnp.int32),
        pltpu.VMEM((EPT,), jnp.float32),
        pltpu.SemaphoreType.DMA,
        pltpu.SemaphoreType.DMA,
    ],
)
def _edge_dots(g_hbm, psrc_hbm, pdst_hbm, nsrc_hbm, ndst_hbm,
               pos_out, neg_out,
               uidx, vidx, widx0, widx1, words0, words1, scores,
               sem0, sem1):
    cid = lax.axis_index("c")
    sid = lax.axis_index("s")
    wid = _worker(cid, sid)

    for src_hbm, dst_hbm, out_hbm in ((psrc_hbm, pdst_hbm, pos_out),
                                      (nsrc_hbm, ndst_hbm, neg_out)):
        pltpu.sync_copy(src_hbm.at[wid], uidx)
        pltpu.sync_copy(dst_hbm.at[wid], vidx)

        def windex(c, widx):
            def grp(g, carry):
                u = uidx[pl.ds(c * K + g * 16, 16)]
                v = vidx[pl.ds(c * K + g * 16, 16)]
                widx[pl.ds(g * 16, 16)] = (
                    lax.shift_right_logical(u, 1) * N_PAD + v)
                return carry
            lax.fori_loop(0, K // 16, grp, 0, unroll=True)

        def gather(widx, words, sem):
            pltpu.async_copy(g_hbm.at[widx], words, sem)

        def wait(widx, words, sem):
            pltpu.make_async_copy(g_hbm.at[widx], words, sem).wait()

        def consume(c, words):
            def grp(g, carry):
                w = words[pl.ds(g * 16, 16)]
                u = uidx[pl.ds(c * K + g * 16, 16)]
                b = plsc.bitcast(w, jnp.bfloat16)
                lo, hi = plsc.unpack(b, format=plsc.PackFormat.INTERLEAVED)
                odd = lax.bitwise_and(u, 1) == 1
                scores[pl.ds(c * K + g * 16, 16)] = jnp.where(odd, hi, lo)
                return carry
            lax.fori_loop(0, K // 16, grp, 0, unroll=True)

        windex(0, widx0)
        gather(widx0, words0, sem0)

        def pair(i, carry):
            c0 = 2 * i
            windex(c0 + 1, widx1)
            wait(widx0, words0, sem0)
            gather(widx1, words1, sem1)
            consume(c0, words0)
            windex(c0 + 2, widx0)
            wait(widx1, words1, sem1)
            gather(widx0, words0, sem0)
            consume(c0 + 1, words1)
            return carry

        lax.fori_loop(0, NPAIR, pair, 0)
        wait(widx0, words0, sem0)
        consume(NCH - 1, words0)
        pltpu.sync_copy(scores, out_hbm.at[pl.ds(wid * EPT, EPT)])


def kernel(x, pos_edge_index, neg_edge_index, W_self1, W_neigh1, b1,
           W_self2, W_neigh2, b2):
    psrc = pos_edge_index[0].reshape(NW, EPT)
    pdst3 = pos_edge_index[1].reshape(NW, NCH, K)
    pdst = pos_edge_index[1].reshape(NW, EPT)
    nsrc = neg_edge_index[0].reshape(NW, EPT)
    ndst = neg_edge_index[1].reshape(NW, EPT)
    znd = jnp.zeros((N_PAD, D), jnp.float32)
    zn = jnp.zeros((N_PAD,), jnp.float32)

    agg1 = _seg(x, psrc, pdst3, znd)
    if isinstance(agg1, (list, tuple)):
        agg1 = agg1[0]
    degp = _deg_kernel(pdst, zn)
    degp = degp.T
    h1 = _dense(x, agg1[0], agg1[1], degp, W_self1, W_neigh1, b1, relu=True)
    agg2 = _seg(h1, psrc, pdst3, znd)
    if isinstance(agg2, (list, tuple)):
        agg2 = agg2[0]
    h2 = _dense(h1, agg2[0], agg2[1], degp, W_self2, W_neigh2, b2,
                relu=False)
    h2p = jnp.pad(h2, ((0, N_PAD - N), (0, 0)))
    gp = _gram(h2.reshape(N // 2, 2, D), h2p)
    pos_s, neg_s = _edge_dots(gp, psrc, pdst, nsrc, ndst)
    return pos_s.reshape(E, 1), neg_s.reshape(E, 1)
